# baseline (speedup 1.0000x reference)
import functools

import jax
import jax.numpy as jnp
from jax import lax
from jax.experimental import pallas as pl
from jax.experimental.pallas import tpu as pltpu
from jax.experimental.pallas import tpu_sc as plsc

N = 10000
E = 320000
D = 128
H = 256
A = 64

NT = 32
EPT = E // NT
NVREG = EPT // 16
NB = D // 16
NEG = -1e30

BVREG = 25
NBATCH = NVREG // BVREG
TRASH = EPT + 16

_GDN = lax.GatherDimensionNumbers(offset_dims=(), collapsed_slice_dims=(0,),
                                  start_index_map=(0,))


def _lane_perm(v, idx):
    return lax.gather(v, idx[:, None], _GDN, (1,),
                      mode=lax.GatherScatterMode.PROMISE_IN_BOUNDS)


def _tree(op, v):
    ii = lax.broadcasted_iota(jnp.int32, (16,), 0)
    for s in (8, 4, 2, 1):
        v = op(v, _lane_perm(v, ii ^ s))
    return v


def _splat(x):
    return jnp.full((16,), x, jnp.float32)


def _sc_body(edge_hbm, x_hbm, w_hbm, as_hbm, ad_hbm, part_hbm,
             dstv, idxb, srcb, rows, wpre, attv, attd, x0v, wsrcv, wdstv,
             pstg, sem, sem2, sh_w):
    cid = lax.axis_index("c")
    sid = lax.axis_index("s")
    wid = sid * 2 + cid
    base = wid * EPT

    dst_cp = pltpu.async_copy(edge_hbm.at[pl.ds(E + base, EPT)], dstv, sem)

    half = sid % 8
    is_src_half = sid < 8
    pltpu.sync_copy(w_hbm.at[pl.ds(half * 16, 16)], wpre)
    pltpu.sync_copy(as_hbm, attv)
    pltpu.sync_copy(ad_hbm, attd)
    flag = _splat(jnp.where(is_src_half, 1.0, 0.0))
    av = [attv[pl.ds(16 * b, 16)] * flag + attd[pl.ds(16 * b, 16)] * (1.0 - flag)
          for b in range(NB)]
    ii16 = lax.broadcasted_iota(jnp.int32, (16,), 0)
    blk = jnp.zeros((16,), jnp.float32)
    for r in range(16):
        p = wpre[r, pl.ds(0, 16)] * av[0]
        for b in range(1, NB):
            p = p + wpre[r, pl.ds(16 * b, 16)] * av[b]
        d = _tree(jnp.add, p)
        blk = jnp.where(ii16 == r, d, blk)
    pstg[pl.ds(0, 16)] = blk
    pltpu.sync_copy(pstg.at[pl.ds(0, 16)], sh_w.at[sid + 32])
    plsc.subcore_barrier()

    pltpu.sync_copy(sh_w.at[pl.ds(32, NB)], wsrcv)
    pltpu.sync_copy(sh_w.at[pl.ds(32 + NB, NB)], wdstv)
    pltpu.sync_copy(x_hbm.at[0], x0v)
    p = x0v[pl.ds(0, 16)] * wdstv[0, pl.ds(0, 16)]
    for b in range(1, NB):
        p = p + x0v[pl.ds(16 * b, 16)] * wdstv[b, pl.ds(0, 16)]
    adv = _tree(jnp.add, p)
    ws = [wsrcv[b, pl.ds(0, 16)] for b in range(NB)]

    dst_cp.wait()
    idxb[pl.ds(0, 16)] = jnp.zeros((16,), jnp.int32)
    cnt0 = jnp.where(wid == 0, 1, 0)

    def compact_vreg(off, cnt):
        dv = dstv[pl.ds(off, 16)]
        mn = _tree(jnp.minimum, dv)[0]

        def vslow(cnt):
            pltpu.sync_copy(edge_hbm.at[pl.ds(base + off, 16)], srcb)
            sv = srcb[pl.ds(0, 16)]
            for l in range(16):
                pred = dv[l] == 0
                tgt = jnp.where(pred, cnt, TRASH)
                idxb[pl.ds(tgt, 16)] = jnp.full((16,), sv[l], jnp.int32)
                cnt = cnt + jnp.where(pred, 1, 0)
            return cnt

        return lax.cond(mn == 0, vslow, lambda c: c, cnt)

    def batch_body(bi, cnt):
        off0 = bi * (BVREG * 16)
        bmin = dstv[pl.ds(off0, 16)]
        for j in range(1, BVREG):
            bmin = jnp.minimum(bmin, dstv[pl.ds(off0 + j * 16, 16)])
        mn = _tree(jnp.minimum, bmin)[0]

        def slow(cnt):
            def per_vreg(j, cnt):
                return compact_vreg(off0 + j * 16, cnt)
            return lax.fori_loop(0, BVREG, per_vreg, cnt)

        return lax.cond(mn == 0, slow, lambda c: c, cnt)

    cnt = lax.fori_loop(0, NBATCH, batch_body, cnt0)

    minf = jnp.full((16,), NEG, jnp.float32)
    zf = jnp.zeros((16,), jnp.float32)
    carry0 = (minf, zf) + tuple(zf for _ in range(NB))
    nch = (cnt + 15) // 16

    def chunk_body(c, carry):
        m, s = carry[0], carry[1]
        acc = list(carry[2:])
        iv = idxb[pl.ds(c * 16, 16)]
        pltpu.async_copy(x_hbm.at[iv], rows, sem).wait()
        for r in range(16):
            xr = [rows[r, pl.ds(16 * b, 16)] for b in range(NB)]
            p = xr[0] * ws[0]
            for b in range(1, NB):
                p = p + xr[b] * ws[b]
            ev = _tree(jnp.add, p) + adv
            ev = jnp.where(ev > 0, ev, 0.2 * ev)
            valid = (c * 16 + r) < cnt
            vf = _splat(jnp.where(valid, 1.0, 0.0))
            new_m = jnp.maximum(m, ev)
            scale = jnp.exp(m - new_m)
            wgt = jnp.exp(ev - new_m) * vf
            s = s * scale + wgt
            for b in range(NB):
                acc[b] = acc[b] * scale + xr[b] * wgt
            m = new_m
        return (m, s) + tuple(acc)

    carry = lax.fori_loop(0, nch, chunk_body, carry0)

    for b in range(NB):
        pstg[pl.ds(16 * b, 16)] = carry[2 + b]
    pstg[pl.ds(D, 16)] = carry[0]
    pstg[pl.ds(D + 16, 16)] = carry[1]
    pltpu.sync_copy(pstg, part_hbm.at[wid])


_sc_call = functools.partial(
    pl.kernel,
    out_type=jax.ShapeDtypeStruct((NT, 2 * D), jnp.float32),
    mesh=plsc.VectorSubcoreMesh(core_axis_name="c", subcore_axis_name="s",
                                num_cores=2),
    scratch_types=[
        pltpu.VMEM((EPT,), jnp.int32),
        pltpu.VMEM((EPT + 48,), jnp.int32),
        pltpu.VMEM((16,), jnp.int32),
        pltpu.VMEM((16, D), jnp.float32),
        pltpu.VMEM((16, D), jnp.float32),
        pltpu.VMEM((D,), jnp.float32),
        pltpu.VMEM((D,), jnp.float32),
        pltpu.VMEM((D,), jnp.float32),
        pltpu.VMEM((NB, 16), jnp.float32),
        pltpu.VMEM((NB, 16), jnp.float32),
        pltpu.VMEM((2 * D,), jnp.float32),
        pltpu.SemaphoreType.DMA,
        pltpu.SemaphoreType.DMA,
        pltpu.VMEM_SHARED((NT // 2 + 32, 16), jnp.float32),
    ],
)(_sc_body)


def _post_body(p_ref, w_ref, bg_ref, w1_ref, b1_ref, w2_ref, b2_ref, out_ref):
    P = p_ref[:, :]
    accm = P[:, 0:D]
    mcol = P[:, D:D + 1]
    scol = P[:, D + 16:D + 17]
    m = jnp.max(mcol)
    wt = jnp.exp(mcol - m)
    s = jnp.sum(scol * wt)
    vec = jnp.sum(accm * wt, axis=0, keepdims=True)
    vecn = vec / (s + 1e-16)
    dn = (((1,), (0,)), ((), ()))
    h0 = lax.dot_general(vecn, w_ref[:, :], dn,
                         preferred_element_type=jnp.float32) + bg_ref[:, :]
    h0 = jnp.maximum(h0, 0.0)
    h1 = lax.dot_general(h0, w1_ref[:, :], dn,
                         preferred_element_type=jnp.float32) + b1_ref[:, :]
    h1 = jnp.maximum(h1, 0.0)
    q = lax.dot_general(h1, w2_ref[:, :], dn,
                        preferred_element_type=jnp.float32) + b2_ref[:, :]
    out_ref[:, :] = q


def kernel(x, edge_index, W_gat, att_src, att_dst, b_gat, W1, b1, W2, b2):
    ei = edge_index.astype(jnp.int32).reshape(2 * E)
    part = _sc_call(ei, x.astype(jnp.float32), W_gat, att_src, att_dst)
    q = pl.pallas_call(
        _post_body,
        out_shape=jax.ShapeDtypeStruct((1, A), jnp.float32),
    )(part, W_gat, b_gat.reshape(1, D), W1, b1.reshape(1, H),
      W2, b2.reshape(1, A))
    return q.reshape(A)


# speedup vs baseline: 1.0603x; 1.0603x over previous
import functools

import jax
import jax.numpy as jnp
from jax import lax
from jax.experimental import pallas as pl
from jax.experimental.pallas import tpu as pltpu
from jax.experimental.pallas import tpu_sc as plsc

N = 10000
E = 320000
D = 128
H = 256
A = 64

NT = 32
EPT = E // NT
NVREG = EPT // 16
NB = D // 16
NEG = -1e30

BVREG = 25
NBATCH = NVREG // BVREG
TRASH = EPT + 16

_GDN = lax.GatherDimensionNumbers(offset_dims=(), collapsed_slice_dims=(0,),
                                  start_index_map=(0,))


def _lane_perm(v, idx):
    return lax.gather(v, idx[:, None], _GDN, (1,),
                      mode=lax.GatherScatterMode.PROMISE_IN_BOUNDS)


def _tree(op, v):
    ii = lax.broadcasted_iota(jnp.int32, (16,), 0)
    for s in (8, 4, 2, 1):
        v = op(v, _lane_perm(v, ii ^ s))
    return v


def _splat(x):
    return jnp.full((16,), x, jnp.float32)


def _sc_body(edge_hbm, x_hbm, w_hbm, as_hbm, ad_hbm, part_hbm,
             dstv, idxb, srcb, rows, wpre, attv, attd, x0v, wsrcv, wdstv,
             pstg, sem, sem2, sem3, sem4, sem5, sh_w):
    cid = lax.axis_index("c")
    sid = lax.axis_index("s")
    wid = sid * 2 + cid
    base = wid * EPT

    half = sid % 8
    is_src_half = sid < 8
    dst_cp = pltpu.async_copy(edge_hbm.at[pl.ds(E + base, EPT)], dstv, sem)
    wpre_cp = pltpu.async_copy(w_hbm.at[pl.ds(half * 16, 16)], wpre, sem2)
    as_cp = pltpu.async_copy(as_hbm, attv, sem3)
    ad_cp = pltpu.async_copy(ad_hbm, attd, sem4)
    x0_cp = pltpu.async_copy(x_hbm.at[0], x0v, sem5)
    as_cp.wait()
    ad_cp.wait()
    wpre_cp.wait()
    flag = _splat(jnp.where(is_src_half, 1.0, 0.0))
    av = [attv[pl.ds(16 * b, 16)] * flag + attd[pl.ds(16 * b, 16)] * (1.0 - flag)
          for b in range(NB)]
    ii16 = lax.broadcasted_iota(jnp.int32, (16,), 0)
    blk = jnp.zeros((16,), jnp.float32)
    for r in range(16):
        p = wpre[r, pl.ds(0, 16)] * av[0]
        for b in range(1, NB):
            p = p + wpre[r, pl.ds(16 * b, 16)] * av[b]
        d = _tree(jnp.add, p)
        blk = jnp.where(ii16 == r, d, blk)
    pstg[pl.ds(0, 16)] = blk
    pltpu.sync_copy(pstg.at[pl.ds(0, 16)], sh_w.at[sid + 32])
    plsc.subcore_barrier()

    pltpu.sync_copy(sh_w.at[pl.ds(32, NB)], wsrcv)
    pltpu.sync_copy(sh_w.at[pl.ds(32 + NB, NB)], wdstv)
    x0_cp.wait()
    p = x0v[pl.ds(0, 16)] * wdstv[0, pl.ds(0, 16)]
    for b in range(1, NB):
        p = p + x0v[pl.ds(16 * b, 16)] * wdstv[b, pl.ds(0, 16)]
    adv = _tree(jnp.add, p)
    ws = [wsrcv[b, pl.ds(0, 16)] for b in range(NB)]

    dst_cp.wait()
    idxb[pl.ds(0, 16)] = jnp.zeros((16,), jnp.int32)
    cnt0 = jnp.where(wid == 0, 1, 0)

    def compact_vreg(off, cnt):
        dv = dstv[pl.ds(off, 16)]
        mn = _tree(jnp.minimum, dv)[0]

        def vslow(cnt):
            pltpu.sync_copy(edge_hbm.at[pl.ds(base + off, 16)], srcb)
            sv = srcb[pl.ds(0, 16)]
            for l in range(16):
                pred = dv[l] == 0
                tgt = jnp.where(pred, cnt, TRASH)
                idxb[pl.ds(tgt, 16)] = jnp.full((16,), sv[l], jnp.int32)
                cnt = cnt + jnp.where(pred, 1, 0)
            return cnt

        return lax.cond(mn == 0, vslow, lambda c: c, cnt)

    def batch_body(bi, cnt):
        off0 = bi * (BVREG * 16)
        bmin = dstv[pl.ds(off0, 16)]
        for j in range(1, BVREG):
            bmin = jnp.minimum(bmin, dstv[pl.ds(off0 + j * 16, 16)])
        mn = _tree(jnp.minimum, bmin)[0]

        def slow(cnt):
            def per_vreg(j, cnt):
                return compact_vreg(off0 + j * 16, cnt)
            return lax.fori_loop(0, BVREG, per_vreg, cnt)

        return lax.cond(mn == 0, slow, lambda c: c, cnt)

    cnt = lax.fori_loop(0, NBATCH, batch_body, cnt0)

    minf = jnp.full((16,), NEG, jnp.float32)
    zf = jnp.zeros((16,), jnp.float32)
    carry0 = (minf, zf) + tuple(zf for _ in range(NB))
    nch = (cnt + 15) // 16

    def chunk_body(c, carry):
        m, s = carry[0], carry[1]
        acc = list(carry[2:])
        iv = idxb[pl.ds(c * 16, 16)]
        pltpu.async_copy(x_hbm.at[iv], rows, sem).wait()
        for r in range(16):
            xr = [rows[r, pl.ds(16 * b, 16)] for b in range(NB)]
            p = xr[0] * ws[0]
            for b in range(1, NB):
                p = p + xr[b] * ws[b]
            ev = _tree(jnp.add, p) + adv
            ev = jnp.where(ev > 0, ev, 0.2 * ev)
            valid = (c * 16 + r) < cnt
            vf = _splat(jnp.where(valid, 1.0, 0.0))
            new_m = jnp.maximum(m, ev)
            scale = jnp.exp(m - new_m)
            wgt = jnp.exp(ev - new_m) * vf
            s = s * scale + wgt
            for b in range(NB):
                acc[b] = acc[b] * scale + xr[b] * wgt
            m = new_m
        return (m, s) + tuple(acc)

    carry = lax.fori_loop(0, nch, chunk_body, carry0)

    for b in range(NB):
        pstg[pl.ds(16 * b, 16)] = carry[2 + b]
    pstg[pl.ds(D, 16)] = carry[0]
    pstg[pl.ds(D + 16, 16)] = carry[1]
    pltpu.sync_copy(pstg, part_hbm.at[wid])


_sc_call = functools.partial(
    pl.kernel,
    out_type=jax.ShapeDtypeStruct((NT, 2 * D), jnp.float32),
    mesh=plsc.VectorSubcoreMesh(core_axis_name="c", subcore_axis_name="s",
                                num_cores=2),
    scratch_types=[
        pltpu.VMEM((EPT,), jnp.int32),
        pltpu.VMEM((EPT + 48,), jnp.int32),
        pltpu.VMEM((16,), jnp.int32),
        pltpu.VMEM((16, D), jnp.float32),
        pltpu.VMEM((16, D), jnp.float32),
        pltpu.VMEM((D,), jnp.float32),
        pltpu.VMEM((D,), jnp.float32),
        pltpu.VMEM((D,), jnp.float32),
        pltpu.VMEM((NB, 16), jnp.float32),
        pltpu.VMEM((NB, 16), jnp.float32),
        pltpu.VMEM((2 * D,), jnp.float32),
        pltpu.SemaphoreType.DMA,
        pltpu.SemaphoreType.DMA,
        pltpu.SemaphoreType.DMA,
        pltpu.SemaphoreType.DMA,
        pltpu.SemaphoreType.DMA,
        pltpu.VMEM_SHARED((NT // 2 + 32, 16), jnp.float32),
    ],
)(_sc_body)


def _post_body(p_ref, w_ref, bg_ref, w1_ref, b1_ref, w2_ref, b2_ref, out_ref):
    P = p_ref[:, :]
    accm = P[:, 0:D]
    mcol = P[:, D:D + 1]
    scol = P[:, D + 16:D + 17]
    m = jnp.max(mcol)
    wt = jnp.exp(mcol - m)
    s = jnp.sum(scol * wt)
    vec = jnp.sum(accm * wt, axis=0, keepdims=True)
    vecn = vec / (s + 1e-16)
    dn = (((1,), (0,)), ((), ()))
    h0 = lax.dot_general(vecn, w_ref[:, :], dn,
                         preferred_element_type=jnp.float32) + bg_ref[:, :]
    h0 = jnp.maximum(h0, 0.0)
    h1 = lax.dot_general(h0, w1_ref[:, :], dn,
                         preferred_element_type=jnp.float32) + b1_ref[:, :]
    h1 = jnp.maximum(h1, 0.0)
    q = lax.dot_general(h1, w2_ref[:, :], dn,
                        preferred_element_type=jnp.float32) + b2_ref[:, :]
    out_ref[:, :] = q


def kernel(x, edge_index, W_gat, att_src, att_dst, b_gat, W1, b1, W2, b2):
    ei = edge_index.astype(jnp.int32).reshape(2 * E)
    part = _sc_call(ei, x.astype(jnp.float32), W_gat, att_src, att_dst)
    q = pl.pallas_call(
        _post_body,
        out_shape=jax.ShapeDtypeStruct((1, A), jnp.float32),
    )(part, W_gat, b_gat.reshape(1, D), W1, b1.reshape(1, H),
      W2, b2.reshape(1, A))
    return q.reshape(A)
